# baseline (speedup 1.0000x reference)
import functools

import jax
import jax.numpy as jnp
from jax import lax
from jax.experimental import pallas as pl
from jax.experimental.pallas import tpu as pltpu
from jax.experimental.pallas import tpu_sc as plsc

_N = 50000
_E = 800000
_B = 64
_H = 32

_NW = 32
_CHUNK = 128
_K = 4
_CPW = 196
_E_PAD = _NW * _CPW * _CHUNK
_N_PAD = 50048
_EBLK = 1000
_NBLK = 1000

_f32 = jnp.float32


def _gelu(v):
    return 0.5 * v * (1.0 + lax.erf(v / jnp.sqrt(2.0).astype(_f32)))


def _mlp_ln(h, W2, b2, gm, bt):
    h = _gelu(jnp.dot(h, W2, preferred_element_type=_f32) + b2)
    mu = jnp.mean(h, axis=-1, keepdims=True)
    var = jnp.mean((h - mu) ** 2, axis=-1, keepdims=True)
    return (h - mu) / jnp.sqrt(var + 1e-5) * gm + bt


def _prep_body(xb, bb, ub, W1x, W1u, ao, uo):
    oh = (bb[...] == lax.broadcasted_iota(jnp.int32, (_NBLK, _B), 1)).astype(_f32)
    uu = jnp.dot(oh, ub[...], preferred_element_type=_f32)
    uo[...] = uu
    ao[...] = (jnp.dot(xb[...], W1x[...], preferred_element_type=_f32)
               + jnp.dot(uu, W1u[...], preferred_element_type=_f32))


def _make_prep(x, batch2d, u, W1x, W1u):
    wspec = lambda shp: pl.BlockSpec(shp, lambda i: (0, 0))
    return pl.pallas_call(
        _prep_body,
        grid=(_N // _NBLK,),
        in_specs=[
            pl.BlockSpec((_NBLK, _H), lambda i: (i, 0)),
            pl.BlockSpec((_NBLK, 1), lambda i: (i, 0)),
            wspec((_B, _H)), wspec((_H, _H)), wspec((_H, _H)),
        ],
        out_specs=[
            pl.BlockSpec((_NBLK, _H), lambda i: (i, 0)),
            pl.BlockSpec((_NBLK, _H), lambda i: (i, 0)),
        ],
        out_shape=[
            jax.ShapeDtypeStruct((_N_PAD, _H), _f32),
            jax.ShapeDtypeStruct((_N_PAD, _H), _f32),
        ],
    )(x, batch2d, u, W1x, W1u)


def _sc_gather(a_tab, x, rowsc, colg, zcnt, onesb):
    mesh = plsc.VectorSubcoreMesh(core_axis_name="c", subcore_axis_name="s")
    rpt = _N_PAD // 16

    @functools.partial(
        pl.kernel,
        out_type=[
            jax.ShapeDtypeStruct((_E_PAD, _H), _f32),
            jax.ShapeDtypeStruct((_E_PAD, _H), _f32),
            jax.ShapeDtypeStruct((_N_PAD, 8), _f32),
            jax.ShapeDtypeStruct((_N_PAD, 8), _f32),
        ],
        mesh=mesh,
        scratch_types=[
            pltpu.VMEM_SHARED((_N_PAD, 8), _f32),
            pltpu.VMEM((_K, _CHUNK), jnp.int32),
            pltpu.VMEM((_K, _CHUNK), jnp.int32),
            pltpu.VMEM((_K * _CHUNK, _H), _f32),
            pltpu.VMEM((_K * _CHUNK, _H), _f32),
            pltpu.VMEM((_CHUNK, 8), _f32),
            pltpu.SemaphoreType.DMA,
        ],
        compiler_params=pltpu.CompilerParams(use_tc_tiling_on_sc=False),
    )
    def k(xub_h, x_h, rowsc_h, colg_h, zcnt_h, ones_h,
          srcue_o, dst_o, c0_o, c1_o,
          sh_cnt, idx_r, idx_c, buf_su, buf_d, ones_v, sem):
        cid = lax.axis_index("c")
        sid = lax.axis_index("s")
        wid = sid * 2 + cid
        t0 = sid * rpt
        pltpu.sync_copy(zcnt_h.at[pl.ds(t0, rpt)], sh_cnt.at[pl.ds(t0, rpt)])
        pltpu.sync_copy(ones_h, ones_v)
        plsc.subcore_barrier()

        def step(g, _):
            cbase = wid * _CPW + g * _K
            pltpu.sync_copy(rowsc_h.at[pl.ds(cbase, _K)], idx_r)
            pltpu.sync_copy(colg_h.at[pl.ds(cbase, _K)], idx_c)
            cps = []
            for j in range(_K):
                cps.append(pltpu.async_copy(
                    xub_h.at[idx_r.at[j]],
                    buf_su.at[pl.ds(j * _CHUNK, _CHUNK)], sem))
                cps.append(pltpu.async_copy(
                    x_h.at[idx_c.at[j]],
                    buf_d.at[pl.ds(j * _CHUNK, _CHUNK)], sem))
            for j in range(_K):
                pltpu.sync_copy(ones_v, sh_cnt.at[idx_r.at[j]], add=True)
            for c in cps:
                c.wait()
            ebase = cbase * _CHUNK
            pltpu.sync_copy(buf_su, srcue_o.at[pl.ds(ebase, _K * _CHUNK)])
            pltpu.sync_copy(buf_d, dst_o.at[pl.ds(ebase, _K * _CHUNK)])
            return ()

        lax.fori_loop(0, _CPW // _K, step, ())
        plsc.subcore_barrier()

        @pl.when(cid == 0)
        def _():
            pltpu.sync_copy(sh_cnt.at[pl.ds(t0, rpt)], c0_o.at[pl.ds(t0, rpt)])

        @pl.when(cid == 1)
        def _():
            pltpu.sync_copy(sh_cnt.at[pl.ds(t0, rpt)], c1_o.at[pl.ds(t0, rpt)])

    return k(a_tab, x, rowsc, colg, zcnt, onesb)


_SBLK = 1000


def _ln_s(h, Mavg, gm, bt):
    mu = jnp.dot(h, Mavg, preferred_element_type=_f32)
    dev = h - mu
    var = jnp.dot(dev * dev, Mavg, preferred_element_type=_f32)
    return dev / jnp.sqrt(var + 1e-5) * gm + bt


def _stk(ref):
    return jnp.concatenate(
        [ref[pl.Slice(g, _SBLK, 4), :] for g in range(4)], axis=1)


def _edge_body(asb, db, eab,
               eW1d, eW1e, eb1, eW2, eb2, egm, ebt,
               nW1d, nW1o, nb1, nW2, nb2, ngm, nbt, Mavg,
               eo, mo):
    d = _stk(db)
    M = Mavg[...]
    h1 = (_stk(asb)
          + jnp.dot(d, eW1d[...], preferred_element_type=_f32)
          + jnp.dot(_stk(eab), eW1e[...], preferred_element_type=_f32)
          + eb1[...])
    h2 = _gelu(jnp.dot(_gelu(h1), eW2[...], preferred_element_type=_f32)
               + eb2[...])
    edge_out = _ln_s(h2, M, egm[...], ebt[...])
    for g in range(4):
        eo[pl.Slice(g, _SBLK, 4), :] = edge_out[:, g * _H:(g + 1) * _H]
    m1 = _gelu(jnp.dot(d, nW1d[...], preferred_element_type=_f32)
               + jnp.dot(edge_out, nW1o[...], preferred_element_type=_f32)
               + nb1[...])
    h3 = _gelu(jnp.dot(m1, nW2[...], preferred_element_type=_f32)
               + nb2[...])
    ms = _ln_s(h3, M, ngm[...], nbt[...])
    for g in range(4):
        mo[pl.Slice(g, _SBLK, 4), :] = ms[:, g * _H:(g + 1) * _H]


def _make_edge(asrc, dstg, edge_attr, ew, nw, Mavg):
    wspec = lambda shp: pl.BlockSpec(shp, lambda i: (0, 0))
    dspec = pl.BlockSpec((4 * _SBLK, _H), lambda i: (i, 0))
    return pl.pallas_call(
        _edge_body,
        grid=(_E // (4 * _SBLK),),
        in_specs=[
            dspec, dspec, dspec,
            wspec((4 * _H, 4 * _H)), wspec((4 * _H, 4 * _H)),
            wspec((1, 4 * _H)),
            wspec((4 * _H, 4 * _H)), wspec((1, 4 * _H)),
            wspec((1, 4 * _H)), wspec((1, 4 * _H)),
            wspec((4 * _H, 4 * _H)), wspec((4 * _H, 4 * _H)),
            wspec((1, 4 * _H)),
            wspec((4 * _H, 4 * _H)), wspec((1, 4 * _H)),
            wspec((1, 4 * _H)), wspec((1, 4 * _H)),
            wspec((4 * _H, 4 * _H)),
        ],
        out_specs=[
            pl.BlockSpec((4 * _SBLK, _H), lambda i: (i, 0)),
            pl.BlockSpec((4 * _SBLK, _H), lambda i: (i, 0)),
        ],
        out_shape=[
            jax.ShapeDtypeStruct((_E, _H), _f32),
            jax.ShapeDtypeStruct((_E_PAD, _H), _f32),
        ],
    )(asrc, dstg, edge_attr, *ew, *nw, Mavg)


def _sc_scatter(m, rows_sc, zsum):
    mesh = plsc.VectorSubcoreMesh(core_axis_name="c", subcore_axis_name="s")
    rpt = _N_PAD // 16

    @functools.partial(
        pl.kernel,
        out_type=[
            jax.ShapeDtypeStruct((_N_PAD, _H), _f32),
            jax.ShapeDtypeStruct((_N_PAD, _H), _f32),
        ],
        mesh=mesh,
        scratch_types=[
            pltpu.VMEM_SHARED((_N_PAD, _H), _f32),
            pltpu.VMEM((_K, _CHUNK), jnp.int32),
            pltpu.VMEM((_K * _CHUNK, _H), _f32),
        ],
        compiler_params=pltpu.CompilerParams(use_tc_tiling_on_sc=False),
    )
    def k(m_h, rows_h, zsum_h,
          s0_o, s1_o,
          sh_sum, idx_v, mval):
        cid = lax.axis_index("c")
        sid = lax.axis_index("s")
        wid = sid * 2 + cid
        t0 = sid * rpt
        pltpu.sync_copy(zsum_h.at[pl.ds(t0, rpt)], sh_sum.at[pl.ds(t0, rpt)])
        plsc.subcore_barrier()

        def step(g, _):
            cbase = wid * _CPW + g * _K
            pltpu.sync_copy(rows_h.at[pl.ds(cbase, _K)], idx_v)
            pltpu.sync_copy(m_h.at[pl.ds(cbase * _CHUNK, _K * _CHUNK)], mval)
            for j in range(_K):
                pltpu.sync_copy(mval.at[pl.ds(j * _CHUNK, _CHUNK)],
                                sh_sum.at[idx_v.at[j]], add=True)
            return ()

        lax.fori_loop(0, _CPW // _K, step, ())
        plsc.subcore_barrier()

        @pl.when(cid == 0)
        def _():
            pltpu.sync_copy(sh_sum.at[pl.ds(t0, rpt)], s0_o.at[pl.ds(t0, rpt)])

        @pl.when(cid == 1)
        def _():
            pltpu.sync_copy(sh_sum.at[pl.ds(t0, rpt)], s1_o.at[pl.ds(t0, rpt)])

    return k(m, rows_sc, zsum)


def _node_body(p0, p1, c0, c1, uub, bt_, ub,
               W1a, W1b, b1, W2, b2, gm, bt,
               xo, gso, gco):
    i = pl.program_id(0)
    cnt = jnp.maximum(c0[:, 0:1] + c1[:, 0:1], 1.0)
    agg = (p0[...] + p1[...]) / cnt
    uu = uub[...]
    h1 = _gelu(jnp.dot(agg, W1a[...], preferred_element_type=_f32)
               + jnp.dot(uu, W1b[...], preferred_element_type=_f32)
               + b1[...])
    xn = _mlp_ln(h1, W2[...], b2[...], gm[...], bt[...])
    xo[...] = xn
    bt_row = bt_[...].reshape(1, _NBLK)
    ohT = (lax.broadcasted_iota(jnp.int32, (_B, _NBLK), 0) == bt_row).astype(_f32)
    gp = jnp.dot(ohT, xn, preferred_element_type=_f32)
    gc = jnp.sum(ohT, axis=1, keepdims=True)

    @pl.when(i == 0)
    def _():
        gso[...] = gp
        gco[...] = gc

    @pl.when(i > 0)
    def _():
        gso[...] += gp
        gco[...] += gc


def _make_node(p0, p1, c0, c1, uu, batchT, u, nw):
    wspec = lambda shp: pl.BlockSpec(shp, lambda i: (0, 0))
    return pl.pallas_call(
        _node_body,
        grid=(_N // _NBLK,),
        in_specs=[
            pl.BlockSpec((_NBLK, _H), lambda i: (i, 0)),
            pl.BlockSpec((_NBLK, _H), lambda i: (i, 0)),
            pl.BlockSpec((_NBLK, 8), lambda i: (i, 0)),
            pl.BlockSpec((_NBLK, 8), lambda i: (i, 0)),
            pl.BlockSpec((_NBLK, _H), lambda i: (i, 0)),
            pl.BlockSpec((1, 1, _NBLK), lambda i: (i, 0, 0)),
            wspec((_B, _H)),
            wspec((_H, _H)), wspec((_H, _H)), wspec((1, _H)),
            wspec((_H, _H)), wspec((1, _H)), wspec((1, _H)), wspec((1, _H)),
        ],
        out_specs=[
            pl.BlockSpec((_NBLK, _H), lambda i: (i, 0)),
            pl.BlockSpec((_B, _H), lambda i: (0, 0)),
            pl.BlockSpec((_B, 1), lambda i: (0, 0)),
        ],
        out_shape=[
            jax.ShapeDtypeStruct((_N, _H), _f32),
            jax.ShapeDtypeStruct((_B, _H), _f32),
            jax.ShapeDtypeStruct((_B, 1), _f32),
        ],
    )(p0, p1, c0, c1, uu, batchT, u, *nw)


def _glob_body(ub, gsb, gcb, W1a, W1b, b1, W2, b2, gm, bt, uo):
    mean = gsb[...] / jnp.maximum(gcb[...], 1.0)
    h1 = _gelu(jnp.dot(ub[...], W1a[...], preferred_element_type=_f32)
               + jnp.dot(mean, W1b[...], preferred_element_type=_f32)
               + b1[...])
    uo[...] = _mlp_ln(h1, W2[...], b2[...], gm[...], bt[...])


def _make_glob(u, gs, gc, gw):
    return pl.pallas_call(
        _glob_body,
        out_shape=jax.ShapeDtypeStruct((_B, _H), _f32),
    )(u, gs, gc, *gw)


def kernel(x, edge_index, edge_attr, u, batch,
           e_W1, e_b1, e_W2, e_b2, e_gm, e_bt,
           n1_W1, n1_b1, n1_W2, n1_b2, n1_gm, n1_bt,
           n2_W1, n2_b1, n2_W2, n2_b2, n2_gm, n2_bt,
           g_W1, g_b1, g_W2, g_b2, g_gm, g_bt):
    row = edge_index[0]
    col = edge_index[1]
    pad = _E_PAD - _E
    colg = jnp.concatenate([col, jnp.zeros((pad,), jnp.int32)]).reshape(-1, _CHUNK)
    rows_sc = jnp.concatenate([row, jnp.full((pad,), _N, jnp.int32)]).reshape(-1, _CHUNK)
    batch2d = batch.reshape(_N, 1)
    batchT = batch.reshape(_N // _NBLK, 1, _NBLK)
    r2 = lambda v: v.reshape(1, _H)

    zcnt = jnp.zeros((_N_PAD, 8), _f32)
    onesb = jnp.ones((_CHUNK, 8), _f32)
    a_tab, uu = _make_prep(x, batch2d, u, e_W1[0:_H], e_W1[3 * _H:4 * _H])
    asrc, dstg, c0, c1 = _sc_gather(a_tab, x, rows_sc, colg, zcnt, onesb)

    eye4 = jnp.eye(4, dtype=_f32)
    bd = lambda w: jnp.kron(eye4, w)
    t4 = lambda v: jnp.tile(v.reshape(1, _H), (1, 4))
    Mavg = jnp.kron(eye4, jnp.full((_H, _H), 1.0 / _H, _f32))

    ew = (bd(e_W1[_H:2 * _H]), bd(e_W1[2 * _H:3 * _H]), t4(e_b1),
          bd(e_W2), t4(e_b2), t4(e_gm), t4(e_bt))
    nw = (bd(n1_W1[0:_H]), bd(n1_W1[_H:2 * _H]), t4(n1_b1),
          bd(n1_W2), t4(n1_b2), t4(n1_gm), t4(n1_bt))
    edge_out, m = _make_edge(asrc, dstg, edge_attr, ew, nw, Mavg)

    zsum = jnp.zeros((_N_PAD, _H), _f32)
    s0, s1 = _sc_scatter(m, rows_sc, zsum)

    n2w = (n2_W1[0:_H], n2_W1[_H:2 * _H], r2(n2_b1),
           n2_W2, r2(n2_b2), r2(n2_gm), r2(n2_bt))
    x_new, gs, gc = _make_node(s0, s1, c0, c1, uu, batchT, u, n2w)

    gw = (g_W1[0:_H], g_W1[_H:2 * _H], r2(g_b1),
          g_W2, r2(g_b2), r2(g_gm), r2(g_bt))
    u_new = _make_glob(u, gs, gc, gw)

    return (x_new, edge_out, u_new)


# speedup vs baseline: 1.6515x; 1.6515x over previous
import functools

import jax
import jax.numpy as jnp
from jax import lax
from jax.experimental import pallas as pl
from jax.experimental.pallas import tpu as pltpu
from jax.experimental.pallas import tpu_sc as plsc

_N = 50000
_E = 800000
_B = 64
_H = 32

_NW = 32
_CHUNK = 128
_K = 7
_CPW = 196
_E_PAD = _NW * _CPW * _CHUNK
_N_PAD = 50048
_EBLK = 1000
_NBLK = 1000

_f32 = jnp.float32


def _gelu(v):
    return 0.5 * v * (1.0 + lax.erf(v / jnp.sqrt(2.0).astype(_f32)))


def _mlp_ln(h, W2, b2, gm, bt):
    h = _gelu(jnp.dot(h, W2, preferred_element_type=_f32) + b2)
    mu = jnp.mean(h, axis=-1, keepdims=True)
    var = jnp.mean((h - mu) ** 2, axis=-1, keepdims=True)
    return (h - mu) / jnp.sqrt(var + 1e-5) * gm + bt


def _prep_body(xb, bb, ub, W1x, W1u, ao, uo):
    oh = (bb[...] == lax.broadcasted_iota(jnp.int32, (_NBLK, _B), 1)).astype(_f32)
    uu = jnp.dot(oh, ub[...], preferred_element_type=_f32)
    uo[...] = uu
    ao[...] = (jnp.dot(xb[...], W1x[...], preferred_element_type=_f32)
               + jnp.dot(uu, W1u[...], preferred_element_type=_f32))


def _make_prep(x, batch2d, u, W1x, W1u):
    wspec = lambda shp: pl.BlockSpec(shp, lambda i: (0, 0))
    return pl.pallas_call(
        _prep_body,
        grid=(_N // _NBLK,),
        in_specs=[
            pl.BlockSpec((_NBLK, _H), lambda i: (i, 0)),
            pl.BlockSpec((_NBLK, 1), lambda i: (i, 0)),
            wspec((_B, _H)), wspec((_H, _H)), wspec((_H, _H)),
        ],
        out_specs=[
            pl.BlockSpec((_NBLK, _H), lambda i: (i, 0)),
            pl.BlockSpec((_NBLK, _H), lambda i: (i, 0)),
        ],
        out_shape=[
            jax.ShapeDtypeStruct((_N_PAD, _H), _f32),
            jax.ShapeDtypeStruct((_N_PAD, _H), _f32),
        ],
    )(x, batch2d, u, W1x, W1u)


def _sc_gather(a_tab, x, rowsc, colg, zcnt, onesb):
    mesh = plsc.VectorSubcoreMesh(core_axis_name="c", subcore_axis_name="s")
    rpt = _N_PAD // 16

    @functools.partial(
        pl.kernel,
        out_type=[
            jax.ShapeDtypeStruct((_E_PAD, _H), _f32),
            jax.ShapeDtypeStruct((_E_PAD, _H), _f32),
            jax.ShapeDtypeStruct((_N_PAD, 8), _f32),
            jax.ShapeDtypeStruct((_N_PAD, 8), _f32),
        ],
        mesh=mesh,
        scratch_types=[
            pltpu.VMEM_SHARED((_N_PAD, 8), _f32),
            pltpu.VMEM((_K, _CHUNK), jnp.int32),
            pltpu.VMEM((_K, _CHUNK), jnp.int32),
            pltpu.VMEM((_K * _CHUNK, _H), _f32),
            pltpu.VMEM((_K * _CHUNK, _H), _f32),
            pltpu.VMEM((_CHUNK, 8), _f32),
            pltpu.SemaphoreType.DMA,
        ],
        compiler_params=pltpu.CompilerParams(use_tc_tiling_on_sc=False),
    )
    def k(xub_h, x_h, rowsc_h, colg_h, zcnt_h, ones_h,
          srcue_o, dst_o, c0_o, c1_o,
          sh_cnt, idx_r, idx_c, buf_su, buf_d, ones_v, sem):
        cid = lax.axis_index("c")
        sid = lax.axis_index("s")
        wid = sid * 2 + cid
        t0 = sid * rpt
        pltpu.sync_copy(zcnt_h.at[pl.ds(t0, rpt)], sh_cnt.at[pl.ds(t0, rpt)])
        pltpu.sync_copy(ones_h, ones_v)
        plsc.subcore_barrier()

        def step(g, _):
            cbase = wid * _CPW + g * _K
            pltpu.sync_copy(rowsc_h.at[pl.ds(cbase, _K)], idx_r)
            pltpu.sync_copy(colg_h.at[pl.ds(cbase, _K)], idx_c)
            cps = []
            for j in range(_K):
                cps.append(pltpu.async_copy(
                    xub_h.at[idx_r.at[j]],
                    buf_su.at[pl.ds(j * _CHUNK, _CHUNK)], sem))
                cps.append(pltpu.async_copy(
                    x_h.at[idx_c.at[j]],
                    buf_d.at[pl.ds(j * _CHUNK, _CHUNK)], sem))
            for j in range(_K):
                pltpu.sync_copy(ones_v, sh_cnt.at[idx_r.at[j]], add=True)
            for c in cps:
                c.wait()
            ebase = cbase * _CHUNK
            pltpu.sync_copy(buf_su, srcue_o.at[pl.ds(ebase, _K * _CHUNK)])
            pltpu.sync_copy(buf_d, dst_o.at[pl.ds(ebase, _K * _CHUNK)])
            return ()

        lax.fori_loop(0, _CPW // _K, step, ())
        plsc.subcore_barrier()

        @pl.when(cid == 0)
        def _():
            pltpu.sync_copy(sh_cnt.at[pl.ds(t0, rpt)], c0_o.at[pl.ds(t0, rpt)])

        @pl.when(cid == 1)
        def _():
            pltpu.sync_copy(sh_cnt.at[pl.ds(t0, rpt)], c1_o.at[pl.ds(t0, rpt)])

    return k(a_tab, x, rowsc, colg, zcnt, onesb)


_SBLK = 1000


def _ln_s(h, Mavg, gm, bt):
    mu = jnp.dot(h, Mavg, preferred_element_type=_f32)
    dev = h - mu
    var = jnp.dot(dev * dev, Mavg, preferred_element_type=_f32)
    return dev / jnp.sqrt(var + 1e-5) * gm + bt


def _edge_body(asb, db, eab,
               eW1d, eW1e, eb1, eW2, eb2, egm, ebt,
               nW1d, nW1o, nb1, nW2, nb2, ngm, nbt, Mavg,
               eo, mo):
    d = db[...]
    M = Mavg[...]
    h1 = (asb[...]
          + jnp.dot(d, eW1d[...], preferred_element_type=_f32)
          + jnp.dot(eab[...], eW1e[...], preferred_element_type=_f32)
          + eb1[...])
    h2 = _gelu(jnp.dot(_gelu(h1), eW2[...], preferred_element_type=_f32)
               + eb2[...])
    edge_out = _ln_s(h2, M, egm[...], ebt[...])
    eo[...] = edge_out
    m1 = _gelu(jnp.dot(d, nW1d[...], preferred_element_type=_f32)
               + jnp.dot(edge_out, nW1o[...], preferred_element_type=_f32)
               + nb1[...])
    h3 = _gelu(jnp.dot(m1, nW2[...], preferred_element_type=_f32)
               + nb2[...])
    mo[...] = _ln_s(h3, M, ngm[...], nbt[...])


def _make_edge(asrc4, dstg4, edge_attr4, ew, nw, Mavg):
    wspec = lambda shp: pl.BlockSpec(shp, lambda i: (0, 0))
    dspec = pl.BlockSpec((_SBLK, 4 * _H), lambda i: (i, 0))
    return pl.pallas_call(
        _edge_body,
        grid=(_E // (4 * _SBLK),),
        in_specs=[
            dspec, dspec, dspec,
            wspec((4 * _H, 4 * _H)), wspec((4 * _H, 4 * _H)),
            wspec((1, 4 * _H)),
            wspec((4 * _H, 4 * _H)), wspec((1, 4 * _H)),
            wspec((1, 4 * _H)), wspec((1, 4 * _H)),
            wspec((4 * _H, 4 * _H)), wspec((4 * _H, 4 * _H)),
            wspec((1, 4 * _H)),
            wspec((4 * _H, 4 * _H)), wspec((1, 4 * _H)),
            wspec((1, 4 * _H)), wspec((1, 4 * _H)),
            wspec((4 * _H, 4 * _H)),
        ],
        out_specs=[
            pl.BlockSpec((_SBLK, 4 * _H), lambda i: (i, 0)),
            pl.BlockSpec((_SBLK, 4 * _H), lambda i: (i, 0)),
        ],
        out_shape=[
            jax.ShapeDtypeStruct((_E // 4, 4 * _H), _f32),
            jax.ShapeDtypeStruct((_E_PAD // 4, 4 * _H), _f32),
        ],
    )(asrc4, dstg4, edge_attr4, *ew, *nw, Mavg)


def _sc_scatter(m, rows_sc, zsum):
    mesh = plsc.VectorSubcoreMesh(core_axis_name="c", subcore_axis_name="s")
    rpt = _N_PAD // 16

    @functools.partial(
        pl.kernel,
        out_type=[
            jax.ShapeDtypeStruct((_N_PAD, _H), _f32),
            jax.ShapeDtypeStruct((_N_PAD, _H), _f32),
        ],
        mesh=mesh,
        scratch_types=[
            pltpu.VMEM_SHARED((_N_PAD, _H), _f32),
            pltpu.VMEM((_K, _CHUNK), jnp.int32),
            pltpu.VMEM((_K * _CHUNK, _H), _f32),
        ],
        compiler_params=pltpu.CompilerParams(use_tc_tiling_on_sc=False),
    )
    def k(m_h, rows_h, zsum_h,
          s0_o, s1_o,
          sh_sum, idx_v, mval):
        cid = lax.axis_index("c")
        sid = lax.axis_index("s")
        wid = sid * 2 + cid
        t0 = sid * rpt
        pltpu.sync_copy(zsum_h.at[pl.ds(t0, rpt)], sh_sum.at[pl.ds(t0, rpt)])
        plsc.subcore_barrier()

        def step(g, _):
            cbase = wid * _CPW + g * _K
            pltpu.sync_copy(rows_h.at[pl.ds(cbase, _K)], idx_v)
            pltpu.sync_copy(m_h.at[pl.ds(cbase * _CHUNK, _K * _CHUNK)], mval)
            for j in range(_K):
                pltpu.sync_copy(mval.at[pl.ds(j * _CHUNK, _CHUNK)],
                                sh_sum.at[idx_v.at[j]], add=True)
            return ()

        lax.fori_loop(0, _CPW // _K, step, ())
        plsc.subcore_barrier()

        @pl.when(cid == 0)
        def _():
            pltpu.sync_copy(sh_sum.at[pl.ds(t0, rpt)], s0_o.at[pl.ds(t0, rpt)])

        @pl.when(cid == 1)
        def _():
            pltpu.sync_copy(sh_sum.at[pl.ds(t0, rpt)], s1_o.at[pl.ds(t0, rpt)])

    return k(m, rows_sc, zsum)


def _node_body(p0, p1, c0, c1, uub, bt_, ub,
               W1a, W1b, b1, W2, b2, gm, bt,
               xo, gso, gco):
    i = pl.program_id(0)
    cnt = jnp.maximum(c0[:, 0:1] + c1[:, 0:1], 1.0)
    agg = (p0[...] + p1[...]) / cnt
    uu = uub[...]
    h1 = _gelu(jnp.dot(agg, W1a[...], preferred_element_type=_f32)
               + jnp.dot(uu, W1b[...], preferred_element_type=_f32)
               + b1[...])
    xn = _mlp_ln(h1, W2[...], b2[...], gm[...], bt[...])
    xo[...] = xn
    bt_row = bt_[...].reshape(1, _NBLK)
    ohT = (lax.broadcasted_iota(jnp.int32, (_B, _NBLK), 0) == bt_row).astype(_f32)
    gp = jnp.dot(ohT, xn, preferred_element_type=_f32)
    gc = jnp.sum(ohT, axis=1, keepdims=True)

    @pl.when(i == 0)
    def _():
        gso[...] = gp
        gco[...] = gc

    @pl.when(i > 0)
    def _():
        gso[...] += gp
        gco[...] += gc


def _make_node(p0, p1, c0, c1, uu, batchT, u, nw):
    wspec = lambda shp: pl.BlockSpec(shp, lambda i: (0, 0))
    return pl.pallas_call(
        _node_body,
        grid=(_N // _NBLK,),
        in_specs=[
            pl.BlockSpec((_NBLK, _H), lambda i: (i, 0)),
            pl.BlockSpec((_NBLK, _H), lambda i: (i, 0)),
            pl.BlockSpec((_NBLK, 8), lambda i: (i, 0)),
            pl.BlockSpec((_NBLK, 8), lambda i: (i, 0)),
            pl.BlockSpec((_NBLK, _H), lambda i: (i, 0)),
            pl.BlockSpec((1, 1, _NBLK), lambda i: (i, 0, 0)),
            wspec((_B, _H)),
            wspec((_H, _H)), wspec((_H, _H)), wspec((1, _H)),
            wspec((_H, _H)), wspec((1, _H)), wspec((1, _H)), wspec((1, _H)),
        ],
        out_specs=[
            pl.BlockSpec((_NBLK, _H), lambda i: (i, 0)),
            pl.BlockSpec((_B, _H), lambda i: (0, 0)),
            pl.BlockSpec((_B, 1), lambda i: (0, 0)),
        ],
        out_shape=[
            jax.ShapeDtypeStruct((_N, _H), _f32),
            jax.ShapeDtypeStruct((_B, _H), _f32),
            jax.ShapeDtypeStruct((_B, 1), _f32),
        ],
    )(p0, p1, c0, c1, uu, batchT, u, *nw)


def _glob_body(ub, gsb, gcb, W1a, W1b, b1, W2, b2, gm, bt, uo):
    mean = gsb[...] / jnp.maximum(gcb[...], 1.0)
    h1 = _gelu(jnp.dot(ub[...], W1a[...], preferred_element_type=_f32)
               + jnp.dot(mean, W1b[...], preferred_element_type=_f32)
               + b1[...])
    uo[...] = _mlp_ln(h1, W2[...], b2[...], gm[...], bt[...])


def _make_glob(u, gs, gc, gw):
    return pl.pallas_call(
        _glob_body,
        out_shape=jax.ShapeDtypeStruct((_B, _H), _f32),
    )(u, gs, gc, *gw)


def kernel(x, edge_index, edge_attr, u, batch,
           e_W1, e_b1, e_W2, e_b2, e_gm, e_bt,
           n1_W1, n1_b1, n1_W2, n1_b2, n1_gm, n1_bt,
           n2_W1, n2_b1, n2_W2, n2_b2, n2_gm, n2_bt,
           g_W1, g_b1, g_W2, g_b2, g_gm, g_bt):
    row = edge_index[0]
    col = edge_index[1]
    pad = _E_PAD - _E
    colg = jnp.concatenate([col, jnp.zeros((pad,), jnp.int32)]).reshape(-1, _CHUNK)
    rows_sc = jnp.concatenate([row, jnp.full((pad,), _N, jnp.int32)]).reshape(-1, _CHUNK)
    batch2d = batch.reshape(_N, 1)
    batchT = batch.reshape(_N // _NBLK, 1, _NBLK)
    r2 = lambda v: v.reshape(1, _H)

    zcnt = jnp.zeros((_N_PAD, 8), _f32)
    onesb = jnp.ones((_CHUNK, 8), _f32)
    a_tab, uu = _make_prep(x, batch2d, u, e_W1[0:_H], e_W1[3 * _H:4 * _H])
    asrc, dstg, c0, c1 = _sc_gather(a_tab, x, rows_sc, colg, zcnt, onesb)

    eye4 = jnp.eye(4, dtype=_f32)
    bd = lambda w: jnp.kron(eye4, w)
    t4 = lambda v: jnp.tile(v.reshape(1, _H), (1, 4))
    Mavg = jnp.kron(eye4, jnp.full((_H, _H), 1.0 / _H, _f32))

    ew = (bd(e_W1[_H:2 * _H]), bd(e_W1[2 * _H:3 * _H]), t4(e_b1),
          bd(e_W2), t4(e_b2), t4(e_gm), t4(e_bt))
    nw = (bd(n1_W1[0:_H]), bd(n1_W1[_H:2 * _H]), t4(n1_b1),
          bd(n1_W2), t4(n1_b2), t4(n1_gm), t4(n1_bt))
    asrc4 = asrc.reshape(_E_PAD // 4, 4 * _H)
    dstg4 = dstg.reshape(_E_PAD // 4, 4 * _H)
    ea4 = edge_attr.reshape(_E // 4, 4 * _H)
    eo4, m4 = _make_edge(asrc4, dstg4, ea4, ew, nw, Mavg)
    edge_out = eo4.reshape(_E, _H)
    m = m4.reshape(_E_PAD, _H)

    zsum = jnp.zeros((_N_PAD, _H), _f32)
    s0, s1 = _sc_scatter(m, rows_sc, zsum)

    n2w = (n2_W1[0:_H], n2_W1[_H:2 * _H], r2(n2_b1),
           n2_W2, r2(n2_b2), r2(n2_gm), r2(n2_bt))
    x_new, gs, gc = _make_node(s0, s1, c0, c1, uu, batchT, u, n2w)

    gw = (g_W1[0:_H], g_W1[_H:2 * _H], r2(g_b1),
          g_W2, r2(g_b2), r2(g_gm), r2(g_bt))
    u_new = _make_glob(u, gs, gc, gw)

    return (x_new, edge_out, u_new)


# speedup vs baseline: 1.7544x; 1.0623x over previous
import functools

import jax
import jax.numpy as jnp
from jax import lax
from jax.experimental import pallas as pl
from jax.experimental.pallas import tpu as pltpu
from jax.experimental.pallas import tpu_sc as plsc

_N = 50000
_E = 800000
_B = 64
_H = 32

_NW = 32
_CHUNK = 128
_K = 4
_CPW = 196
_E_PAD = _NW * _CPW * _CHUNK
_N_PAD = 50048
_EBLK = 1000
_NBLK = 1000

_f32 = jnp.float32


def _gelu(v):
    return 0.5 * v * (1.0 + lax.erf(v / jnp.sqrt(2.0).astype(_f32)))


def _mlp_ln(h, W2, b2, gm, bt):
    h = _gelu(jnp.dot(h, W2, preferred_element_type=_f32) + b2)
    mu = jnp.mean(h, axis=-1, keepdims=True)
    var = jnp.mean((h - mu) ** 2, axis=-1, keepdims=True)
    return (h - mu) / jnp.sqrt(var + 1e-5) * gm + bt


def _prep_body(xb, bb, ub, W1x, W1u, ao, uo):
    oh = (bb[...] == lax.broadcasted_iota(jnp.int32, (_NBLK, _B), 1)).astype(_f32)
    uu = jnp.dot(oh, ub[...], preferred_element_type=_f32)
    uo[...] = uu
    ao[...] = (jnp.dot(xb[...], W1x[...], preferred_element_type=_f32)
               + jnp.dot(uu, W1u[...], preferred_element_type=_f32))


def _make_prep(x, batch2d, u, W1x, W1u):
    wspec = lambda shp: pl.BlockSpec(shp, lambda i: (0, 0))
    return pl.pallas_call(
        _prep_body,
        grid=(_N // _NBLK,),
        in_specs=[
            pl.BlockSpec((_NBLK, _H), lambda i: (i, 0)),
            pl.BlockSpec((_NBLK, 1), lambda i: (i, 0)),
            wspec((_B, _H)), wspec((_H, _H)), wspec((_H, _H)),
        ],
        out_specs=[
            pl.BlockSpec((_NBLK, _H), lambda i: (i, 0)),
            pl.BlockSpec((_NBLK, _H), lambda i: (i, 0)),
        ],
        out_shape=[
            jax.ShapeDtypeStruct((_N_PAD, _H), _f32),
            jax.ShapeDtypeStruct((_N_PAD, _H), _f32),
        ],
    )(x, batch2d, u, W1x, W1u)


def _sc_gather(a_tab, x, rowsc, colg, zcnt, onesb):
    mesh = plsc.VectorSubcoreMesh(core_axis_name="c", subcore_axis_name="s")
    rpt = _N_PAD // 16

    @functools.partial(
        pl.kernel,
        out_type=[
            jax.ShapeDtypeStruct((_E_PAD, _H), _f32),
            jax.ShapeDtypeStruct((_E_PAD, _H), _f32),
            jax.ShapeDtypeStruct((_N_PAD, 8), _f32),
            jax.ShapeDtypeStruct((_N_PAD, 8), _f32),
        ],
        mesh=mesh,
        scratch_types=[
            pltpu.VMEM_SHARED((_N_PAD, 8), _f32),
            pltpu.VMEM((_K, _CHUNK), jnp.int32),
            pltpu.VMEM((_K, _CHUNK), jnp.int32),
            pltpu.VMEM((_K * _CHUNK, _H), _f32),
            pltpu.VMEM((_K * _CHUNK, _H), _f32),
            pltpu.VMEM((_CHUNK, 8), _f32),
            pltpu.SemaphoreType.DMA,
        ],
        compiler_params=pltpu.CompilerParams(use_tc_tiling_on_sc=False),
    )
    def k(xub_h, x_h, rowsc_h, colg_h, zcnt_h, ones_h,
          srcue_o, dst_o, c0_o, c1_o,
          sh_cnt, idx_r, idx_c, buf_su, buf_d, ones_v, sem):
        cid = lax.axis_index("c")
        sid = lax.axis_index("s")
        wid = sid * 2 + cid
        t0 = sid * rpt
        pltpu.sync_copy(zcnt_h.at[pl.ds(t0, rpt)], sh_cnt.at[pl.ds(t0, rpt)])
        pltpu.sync_copy(ones_h, ones_v)
        plsc.subcore_barrier()

        def step(g, _):
            cbase = wid * _CPW + g * _K
            pltpu.sync_copy(rowsc_h.at[pl.ds(cbase, _K)], idx_r)
            pltpu.sync_copy(colg_h.at[pl.ds(cbase, _K)], idx_c)
            cps = []
            for j in range(_K):
                cps.append(pltpu.async_copy(
                    xub_h.at[idx_r.at[j]],
                    buf_su.at[pl.ds(j * _CHUNK, _CHUNK)], sem))
                cps.append(pltpu.async_copy(
                    x_h.at[idx_c.at[j]],
                    buf_d.at[pl.ds(j * _CHUNK, _CHUNK)], sem))
            for j in range(_K):
                pltpu.sync_copy(ones_v, sh_cnt.at[idx_r.at[j]], add=True)
            for c in cps:
                c.wait()
            ebase = cbase * _CHUNK
            pltpu.sync_copy(buf_su, srcue_o.at[pl.ds(ebase, _K * _CHUNK)])
            pltpu.sync_copy(buf_d, dst_o.at[pl.ds(ebase, _K * _CHUNK)])
            return ()

        lax.fori_loop(0, _CPW // _K, step, ())
        plsc.subcore_barrier()

        @pl.when(cid == 0)
        def _():
            pltpu.sync_copy(sh_cnt.at[pl.ds(t0, rpt)], c0_o.at[pl.ds(t0, rpt)])

        @pl.when(cid == 1)
        def _():
            pltpu.sync_copy(sh_cnt.at[pl.ds(t0, rpt)], c1_o.at[pl.ds(t0, rpt)])

    return k(a_tab, x, rowsc, colg, zcnt, onesb)


_SBLK = 2000


def _ln_s(h, Mavg, gm, bt):
    mu = jnp.dot(h, Mavg, preferred_element_type=_f32)
    dev = h - mu
    var = jnp.dot(dev * dev, Mavg, preferred_element_type=_f32)
    return dev / jnp.sqrt(var + 1e-5) * gm + bt


def _edge_body(asb, db, eab,
               eW1d, eW1e, eb1, eW2, eb2, egm, ebt,
               nW1d, nW1o, nb1, nW2, nb2, ngm, nbt, Mavg,
               eo, mo):
    d = db[...]
    M = Mavg[...]
    h1 = (asb[...]
          + jnp.dot(d, eW1d[...], preferred_element_type=_f32)
          + jnp.dot(eab[...], eW1e[...], preferred_element_type=_f32)
          + eb1[...])
    h2 = _gelu(jnp.dot(_gelu(h1), eW2[...], preferred_element_type=_f32)
               + eb2[...])
    edge_out = _ln_s(h2, M, egm[...], ebt[...])
    eo[...] = edge_out
    m1 = _gelu(jnp.dot(d, nW1d[...], preferred_element_type=_f32)
               + jnp.dot(edge_out, nW1o[...], preferred_element_type=_f32)
               + nb1[...])
    h3 = _gelu(jnp.dot(m1, nW2[...], preferred_element_type=_f32)
               + nb2[...])
    mo[...] = _ln_s(h3, M, ngm[...], nbt[...])


def _make_edge(asrc4, dstg4, edge_attr4, ew, nw, Mavg):
    wspec = lambda shp: pl.BlockSpec(shp, lambda i: (0, 0))
    dspec = pl.BlockSpec((_SBLK, 4 * _H), lambda i: (i, 0))
    return pl.pallas_call(
        _edge_body,
        grid=(_E // (4 * _SBLK),),
        in_specs=[
            dspec, dspec, dspec,
            wspec((4 * _H, 4 * _H)), wspec((4 * _H, 4 * _H)),
            wspec((1, 4 * _H)),
            wspec((4 * _H, 4 * _H)), wspec((1, 4 * _H)),
            wspec((1, 4 * _H)), wspec((1, 4 * _H)),
            wspec((4 * _H, 4 * _H)), wspec((4 * _H, 4 * _H)),
            wspec((1, 4 * _H)),
            wspec((4 * _H, 4 * _H)), wspec((1, 4 * _H)),
            wspec((1, 4 * _H)), wspec((1, 4 * _H)),
            wspec((4 * _H, 4 * _H)),
        ],
        out_specs=[
            pl.BlockSpec((_SBLK, 4 * _H), lambda i: (i, 0)),
            pl.BlockSpec((_SBLK, 4 * _H), lambda i: (i, 0)),
        ],
        out_shape=[
            jax.ShapeDtypeStruct((_E // 4, 4 * _H), _f32),
            jax.ShapeDtypeStruct((_E_PAD // 4, 4 * _H), _f32),
        ],
    )(asrc4, dstg4, edge_attr4, *ew, *nw, Mavg)


def _sc_scatter(m, rows_sc, zsum):
    mesh = plsc.VectorSubcoreMesh(core_axis_name="c", subcore_axis_name="s")
    rpt = _N_PAD // 16

    @functools.partial(
        pl.kernel,
        out_type=[
            jax.ShapeDtypeStruct((_N_PAD, _H), _f32),
            jax.ShapeDtypeStruct((_N_PAD, _H), _f32),
        ],
        mesh=mesh,
        scratch_types=[
            pltpu.VMEM_SHARED((_N_PAD, _H), _f32),
            pltpu.VMEM((_K, _CHUNK), jnp.int32),
            pltpu.VMEM((_K * _CHUNK, _H), _f32),
        ],
        compiler_params=pltpu.CompilerParams(use_tc_tiling_on_sc=False),
    )
    def k(m_h, rows_h, zsum_h,
          s0_o, s1_o,
          sh_sum, idx_v, mval):
        cid = lax.axis_index("c")
        sid = lax.axis_index("s")
        wid = sid * 2 + cid
        t0 = sid * rpt
        pltpu.sync_copy(zsum_h.at[pl.ds(t0, rpt)], sh_sum.at[pl.ds(t0, rpt)])
        plsc.subcore_barrier()

        def step(g, _):
            cbase = wid * _CPW + g * _K
            pltpu.sync_copy(rows_h.at[pl.ds(cbase, _K)], idx_v)
            pltpu.sync_copy(m_h.at[pl.ds(cbase * _CHUNK, _K * _CHUNK)], mval)
            for j in range(_K):
                pltpu.sync_copy(mval.at[pl.ds(j * _CHUNK, _CHUNK)],
                                sh_sum.at[idx_v.at[j]], add=True)
            return ()

        lax.fori_loop(0, _CPW // _K, step, ())
        plsc.subcore_barrier()

        @pl.when(cid == 0)
        def _():
            pltpu.sync_copy(sh_sum.at[pl.ds(t0, rpt)], s0_o.at[pl.ds(t0, rpt)])

        @pl.when(cid == 1)
        def _():
            pltpu.sync_copy(sh_sum.at[pl.ds(t0, rpt)], s1_o.at[pl.ds(t0, rpt)])

    return k(m, rows_sc, zsum)


def _node_body(p0, p1, c0, c1, uub, bt_, ub,
               W1a, W1b, b1, W2, b2, gm, bt,
               xo, gso, gco):
    i = pl.program_id(0)
    cnt = jnp.maximum(c0[:, 0:1] + c1[:, 0:1], 1.0)
    agg = (p0[...] + p1[...]) / cnt
    uu = uub[...]
    h1 = _gelu(jnp.dot(agg, W1a[...], preferred_element_type=_f32)
               + jnp.dot(uu, W1b[...], preferred_element_type=_f32)
               + b1[...])
    xn = _mlp_ln(h1, W2[...], b2[...], gm[...], bt[...])
    xo[...] = xn
    bt_row = bt_[...].reshape(1, _NBLK)
    ohT = (lax.broadcasted_iota(jnp.int32, (_B, _NBLK), 0) == bt_row).astype(_f32)
    gp = jnp.dot(ohT, xn, preferred_element_type=_f32)
    gc = jnp.sum(ohT, axis=1, keepdims=True)

    @pl.when(i == 0)
    def _():
        gso[...] = gp
        gco[...] = gc

    @pl.when(i > 0)
    def _():
        gso[...] += gp
        gco[...] += gc


def _make_node(p0, p1, c0, c1, uu, batchT, u, nw):
    wspec = lambda shp: pl.BlockSpec(shp, lambda i: (0, 0))
    return pl.pallas_call(
        _node_body,
        grid=(_N // _NBLK,),
        in_specs=[
            pl.BlockSpec((_NBLK, _H), lambda i: (i, 0)),
            pl.BlockSpec((_NBLK, _H), lambda i: (i, 0)),
            pl.BlockSpec((_NBLK, 8), lambda i: (i, 0)),
            pl.BlockSpec((_NBLK, 8), lambda i: (i, 0)),
            pl.BlockSpec((_NBLK, _H), lambda i: (i, 0)),
            pl.BlockSpec((1, 1, _NBLK), lambda i: (i, 0, 0)),
            wspec((_B, _H)),
            wspec((_H, _H)), wspec((_H, _H)), wspec((1, _H)),
            wspec((_H, _H)), wspec((1, _H)), wspec((1, _H)), wspec((1, _H)),
        ],
        out_specs=[
            pl.BlockSpec((_NBLK, _H), lambda i: (i, 0)),
            pl.BlockSpec((_B, _H), lambda i: (0, 0)),
            pl.BlockSpec((_B, 1), lambda i: (0, 0)),
        ],
        out_shape=[
            jax.ShapeDtypeStruct((_N, _H), _f32),
            jax.ShapeDtypeStruct((_B, _H), _f32),
            jax.ShapeDtypeStruct((_B, 1), _f32),
        ],
    )(p0, p1, c0, c1, uu, batchT, u, *nw)


def _glob_body(ub, gsb, gcb, W1a, W1b, b1, W2, b2, gm, bt, uo):
    mean = gsb[...] / jnp.maximum(gcb[...], 1.0)
    h1 = _gelu(jnp.dot(ub[...], W1a[...], preferred_element_type=_f32)
               + jnp.dot(mean, W1b[...], preferred_element_type=_f32)
               + b1[...])
    uo[...] = _mlp_ln(h1, W2[...], b2[...], gm[...], bt[...])


def _make_glob(u, gs, gc, gw):
    return pl.pallas_call(
        _glob_body,
        out_shape=jax.ShapeDtypeStruct((_B, _H), _f32),
    )(u, gs, gc, *gw)


def kernel(x, edge_index, edge_attr, u, batch,
           e_W1, e_b1, e_W2, e_b2, e_gm, e_bt,
           n1_W1, n1_b1, n1_W2, n1_b2, n1_gm, n1_bt,
           n2_W1, n2_b1, n2_W2, n2_b2, n2_gm, n2_bt,
           g_W1, g_b1, g_W2, g_b2, g_gm, g_bt):
    row = edge_index[0]
    col = edge_index[1]
    pad = _E_PAD - _E
    colg = jnp.concatenate([col, jnp.zeros((pad,), jnp.int32)]).reshape(-1, _CHUNK)
    rows_sc = jnp.concatenate([row, jnp.full((pad,), _N, jnp.int32)]).reshape(-1, _CHUNK)
    batch2d = batch.reshape(_N, 1)
    batchT = batch.reshape(_N // _NBLK, 1, _NBLK)
    r2 = lambda v: v.reshape(1, _H)

    zcnt = jnp.zeros((_N_PAD, 8), _f32)
    onesb = jnp.ones((_CHUNK, 8), _f32)
    a_tab, uu = _make_prep(x, batch2d, u, e_W1[0:_H], e_W1[3 * _H:4 * _H])
    asrc, dstg, c0, c1 = _sc_gather(a_tab, x, rows_sc, colg, zcnt, onesb)

    eye4 = jnp.eye(4, dtype=_f32)
    bd = lambda w: jnp.kron(eye4, w)
    t4 = lambda v: jnp.tile(v.reshape(1, _H), (1, 4))
    Mavg = jnp.kron(eye4, jnp.full((_H, _H), 1.0 / _H, _f32))

    ew = (bd(e_W1[_H:2 * _H]), bd(e_W1[2 * _H:3 * _H]), t4(e_b1),
          bd(e_W2), t4(e_b2), t4(e_gm), t4(e_bt))
    nw = (bd(n1_W1[0:_H]), bd(n1_W1[_H:2 * _H]), t4(n1_b1),
          bd(n1_W2), t4(n1_b2), t4(n1_gm), t4(n1_bt))
    asrc4 = asrc.reshape(_E_PAD // 4, 4 * _H)
    dstg4 = dstg.reshape(_E_PAD // 4, 4 * _H)
    ea4 = edge_attr.reshape(_E // 4, 4 * _H)
    eo4, m4 = _make_edge(asrc4, dstg4, ea4, ew, nw, Mavg)
    edge_out = eo4.reshape(_E, _H)
    m = m4.reshape(_E_PAD, _H)

    zsum = jnp.zeros((_N_PAD, _H), _f32)
    s0, s1 = _sc_scatter(m, rows_sc, zsum)

    n2w = (n2_W1[0:_H], n2_W1[_H:2 * _H], r2(n2_b1),
           n2_W2, r2(n2_b2), r2(n2_gm), r2(n2_bt))
    x_new, gs, gc = _make_node(s0, s1, c0, c1, uu, batchT, u, n2w)

    gw = (g_W1[0:_H], g_W1[_H:2 * _H], r2(g_b1),
          g_W2, r2(g_b2), r2(g_gm), r2(g_bt))
    u_new = _make_glob(u, gs, gc, gw)

    return (x_new, edge_out, u_new)


# speedup vs baseline: 1.7892x; 1.0198x over previous
import functools

import jax
import jax.numpy as jnp
from jax import lax
from jax.experimental import pallas as pl
from jax.experimental.pallas import tpu as pltpu
from jax.experimental.pallas import tpu_sc as plsc

_N = 50000
_E = 800000
_B = 64
_H = 32

_NW = 32
_CHUNK = 128
_K = 4
_CPW = 196
_E_PAD = _NW * _CPW * _CHUNK
_N_PAD = 50048
_EBLK = 1000
_NBLK = 1000

_f32 = jnp.float32


def _gelu(v):
    return 0.5 * v * (1.0 + lax.erf(v / jnp.sqrt(2.0).astype(_f32)))


def _mlp_ln(h, W2, b2, gm, bt):
    h = _gelu(jnp.dot(h, W2, preferred_element_type=_f32) + b2)
    mu = jnp.mean(h, axis=-1, keepdims=True)
    var = jnp.mean((h - mu) ** 2, axis=-1, keepdims=True)
    return (h - mu) / jnp.sqrt(var + 1e-5) * gm + bt


def _prep_body(xb, bb, ub, W1x, W1u, ao, uo):
    oh = (bb[...] == lax.broadcasted_iota(jnp.int32, (_NBLK, _B), 1)).astype(_f32)
    uu = jnp.dot(oh, ub[...], preferred_element_type=_f32)
    uo[...] = uu
    ao[...] = (jnp.dot(xb[...], W1x[...], preferred_element_type=_f32)
               + jnp.dot(uu, W1u[...], preferred_element_type=_f32))


def _make_prep(x, batch2d, u, W1x, W1u):
    wspec = lambda shp: pl.BlockSpec(shp, lambda i: (0, 0))
    return pl.pallas_call(
        _prep_body,
        grid=(_N // _NBLK,),
        in_specs=[
            pl.BlockSpec((_NBLK, _H), lambda i: (i, 0)),
            pl.BlockSpec((_NBLK, 1), lambda i: (i, 0)),
            wspec((_B, _H)), wspec((_H, _H)), wspec((_H, _H)),
        ],
        out_specs=[
            pl.BlockSpec((_NBLK, _H), lambda i: (i, 0)),
            pl.BlockSpec((_NBLK, _H), lambda i: (i, 0)),
        ],
        out_shape=[
            jax.ShapeDtypeStruct((_N_PAD, _H), _f32),
            jax.ShapeDtypeStruct((_N_PAD, _H), _f32),
        ],
    )(x, batch2d, u, W1x, W1u)


def _sc_gather(a_tab, x, rowsc, colg, zcnt, onesb):
    mesh = plsc.VectorSubcoreMesh(core_axis_name="c", subcore_axis_name="s")
    rpt = _N_PAD // 16

    @functools.partial(
        pl.kernel,
        out_type=[
            jax.ShapeDtypeStruct((_E_PAD, _H), _f32),
            jax.ShapeDtypeStruct((_E_PAD, _H), _f32),
            jax.ShapeDtypeStruct((_N_PAD, 8), _f32),
            jax.ShapeDtypeStruct((_N_PAD, 8), _f32),
        ],
        mesh=mesh,
        scratch_types=[
            pltpu.VMEM_SHARED((_N_PAD, 8), _f32),
            pltpu.VMEM((_K, _CHUNK), jnp.int32),
            pltpu.VMEM((_K, _CHUNK), jnp.int32),
            pltpu.VMEM((_K * _CHUNK, _H), _f32),
            pltpu.VMEM((_K * _CHUNK, _H), _f32),
            pltpu.VMEM((_CHUNK, 8), _f32),
            pltpu.SemaphoreType.DMA,
        ],
        compiler_params=pltpu.CompilerParams(use_tc_tiling_on_sc=False),
    )
    def k(xub_h, x_h, rowsc_h, colg_h, zcnt_h, ones_h,
          srcue_o, dst_o, c0_o, c1_o,
          sh_cnt, idx_r, idx_c, buf_su, buf_d, ones_v, sem):
        cid = lax.axis_index("c")
        sid = lax.axis_index("s")
        wid = sid * 2 + cid
        t0 = sid * rpt
        pltpu.sync_copy(zcnt_h.at[pl.ds(t0, rpt)], sh_cnt.at[pl.ds(t0, rpt)])
        pltpu.sync_copy(ones_h, ones_v)
        plsc.subcore_barrier()

        def step(g, _):
            cbase = wid * _CPW + g * _K
            pltpu.sync_copy(rowsc_h.at[pl.ds(cbase, _K)], idx_r)
            pltpu.sync_copy(colg_h.at[pl.ds(cbase, _K)], idx_c)
            cps = []
            for j in range(_K):
                cps.append(pltpu.async_copy(
                    xub_h.at[idx_r.at[j]],
                    buf_su.at[pl.ds(j * _CHUNK, _CHUNK)], sem))
                cps.append(pltpu.async_copy(
                    x_h.at[idx_c.at[j]],
                    buf_d.at[pl.ds(j * _CHUNK, _CHUNK)], sem))
            for j in range(_K):
                pltpu.sync_copy(ones_v, sh_cnt.at[idx_r.at[j]], add=True)
            for c in cps:
                c.wait()
            ebase = cbase * _CHUNK
            pltpu.sync_copy(buf_su, srcue_o.at[pl.ds(ebase, _K * _CHUNK)])
            pltpu.sync_copy(buf_d, dst_o.at[pl.ds(ebase, _K * _CHUNK)])
            return ()

        lax.fori_loop(0, _CPW // _K, step, ())
        plsc.subcore_barrier()

        @pl.when(cid == 0)
        def _():
            pltpu.sync_copy(sh_cnt.at[pl.ds(t0, rpt)], c0_o.at[pl.ds(t0, rpt)])

        @pl.when(cid == 1)
        def _():
            pltpu.sync_copy(sh_cnt.at[pl.ds(t0, rpt)], c1_o.at[pl.ds(t0, rpt)])

    return k(a_tab, x, rowsc, colg, zcnt, onesb)


_SBLK = 4000


def _ln_s(h, Mavg, gm, bt):
    mu = jnp.dot(h, Mavg, preferred_element_type=_f32)
    dev = h - mu
    var = jnp.dot(dev * dev, Mavg, preferred_element_type=_f32)
    return dev / jnp.sqrt(var + 1e-5) * gm + bt


def _edge_body(asb, db, eab,
               eW1d, eW1e, eb1, eW2, eb2, egm, ebt,
               nW1d, nW1o, nb1, nW2, nb2, ngm, nbt, Mavg,
               eo, mo):
    d = db[...]
    M = Mavg[...]
    h1 = (asb[...]
          + jnp.dot(d, eW1d[...], preferred_element_type=_f32)
          + jnp.dot(eab[...], eW1e[...], preferred_element_type=_f32)
          + eb1[...])
    h2 = _gelu(jnp.dot(_gelu(h1), eW2[...], preferred_element_type=_f32)
               + eb2[...])
    edge_out = _ln_s(h2, M, egm[...], ebt[...])
    eo[...] = edge_out
    m1 = _gelu(jnp.dot(d, nW1d[...], preferred_element_type=_f32)
               + jnp.dot(edge_out, nW1o[...], preferred_element_type=_f32)
               + nb1[...])
    h3 = _gelu(jnp.dot(m1, nW2[...], preferred_element_type=_f32)
               + nb2[...])
    mo[...] = _ln_s(h3, M, ngm[...], nbt[...])


def _make_edge(asrc4, dstg4, edge_attr4, ew, nw, Mavg):
    wspec = lambda shp: pl.BlockSpec(shp, lambda i: (0, 0))
    dspec = pl.BlockSpec((_SBLK, 4 * _H), lambda i: (i, 0))
    return pl.pallas_call(
        _edge_body,
        grid=(_E // (4 * _SBLK),),
        in_specs=[
            dspec, dspec, dspec,
            wspec((4 * _H, 4 * _H)), wspec((4 * _H, 4 * _H)),
            wspec((1, 4 * _H)),
            wspec((4 * _H, 4 * _H)), wspec((1, 4 * _H)),
            wspec((1, 4 * _H)), wspec((1, 4 * _H)),
            wspec((4 * _H, 4 * _H)), wspec((4 * _H, 4 * _H)),
            wspec((1, 4 * _H)),
            wspec((4 * _H, 4 * _H)), wspec((1, 4 * _H)),
            wspec((1, 4 * _H)), wspec((1, 4 * _H)),
            wspec((4 * _H, 4 * _H)),
        ],
        out_specs=[
            pl.BlockSpec((_SBLK, 4 * _H), lambda i: (i, 0)),
            pl.BlockSpec((_SBLK, 4 * _H), lambda i: (i, 0)),
        ],
        out_shape=[
            jax.ShapeDtypeStruct((_E // 4, 4 * _H), _f32),
            jax.ShapeDtypeStruct((_E_PAD // 4, 4 * _H), _f32),
        ],
    )(asrc4, dstg4, edge_attr4, *ew, *nw, Mavg)


def _sc_scatter(m, rows_sc, zsum):
    mesh = plsc.VectorSubcoreMesh(core_axis_name="c", subcore_axis_name="s")
    rpt = _N_PAD // 16

    @functools.partial(
        pl.kernel,
        out_type=[
            jax.ShapeDtypeStruct((_N_PAD, _H), _f32),
            jax.ShapeDtypeStruct((_N_PAD, _H), _f32),
        ],
        mesh=mesh,
        scratch_types=[
            pltpu.VMEM_SHARED((_N_PAD, _H), _f32),
            pltpu.VMEM((_K, _CHUNK), jnp.int32),
            pltpu.VMEM((_K * _CHUNK, _H), _f32),
        ],
        compiler_params=pltpu.CompilerParams(use_tc_tiling_on_sc=False),
    )
    def k(m_h, rows_h, zsum_h,
          s0_o, s1_o,
          sh_sum, idx_v, mval):
        cid = lax.axis_index("c")
        sid = lax.axis_index("s")
        wid = sid * 2 + cid
        t0 = sid * rpt
        pltpu.sync_copy(zsum_h.at[pl.ds(t0, rpt)], sh_sum.at[pl.ds(t0, rpt)])
        plsc.subcore_barrier()

        def step(g, _):
            cbase = wid * _CPW + g * _K
            pltpu.sync_copy(rows_h.at[pl.ds(cbase, _K)], idx_v)
            pltpu.sync_copy(m_h.at[pl.ds(cbase * _CHUNK, _K * _CHUNK)], mval)
            for j in range(_K):
                pltpu.sync_copy(mval.at[pl.ds(j * _CHUNK, _CHUNK)],
                                sh_sum.at[idx_v.at[j]], add=True)
            return ()

        lax.fori_loop(0, _CPW // _K, step, ())
        plsc.subcore_barrier()

        @pl.when(cid == 0)
        def _():
            pltpu.sync_copy(sh_sum.at[pl.ds(t0, rpt)], s0_o.at[pl.ds(t0, rpt)])

        @pl.when(cid == 1)
        def _():
            pltpu.sync_copy(sh_sum.at[pl.ds(t0, rpt)], s1_o.at[pl.ds(t0, rpt)])

    return k(m, rows_sc, zsum)


def _node_body(p0, p1, c0, c1, uub, bt_, ub,
               W1a, W1b, b1, W2, b2, gm, bt,
               xo, gso, gco):
    i = pl.program_id(0)
    cnt = jnp.maximum(c0[:, 0:1] + c1[:, 0:1], 1.0)
    agg = (p0[...] + p1[...]) / cnt
    uu = uub[...]
    h1 = _gelu(jnp.dot(agg, W1a[...], preferred_element_type=_f32)
               + jnp.dot(uu, W1b[...], preferred_element_type=_f32)
               + b1[...])
    xn = _mlp_ln(h1, W2[...], b2[...], gm[...], bt[...])
    xo[...] = xn
    bt_row = bt_[...].reshape(1, _NBLK)
    ohT = (lax.broadcasted_iota(jnp.int32, (_B, _NBLK), 0) == bt_row).astype(_f32)
    gp = jnp.dot(ohT, xn, preferred_element_type=_f32)
    gc = jnp.sum(ohT, axis=1, keepdims=True)

    @pl.when(i == 0)
    def _():
        gso[...] = gp
        gco[...] = gc

    @pl.when(i > 0)
    def _():
        gso[...] += gp
        gco[...] += gc


def _make_node(p0, p1, c0, c1, uu, batchT, u, nw):
    wspec = lambda shp: pl.BlockSpec(shp, lambda i: (0, 0))
    return pl.pallas_call(
        _node_body,
        grid=(_N // _NBLK,),
        in_specs=[
            pl.BlockSpec((_NBLK, _H), lambda i: (i, 0)),
            pl.BlockSpec((_NBLK, _H), lambda i: (i, 0)),
            pl.BlockSpec((_NBLK, 8), lambda i: (i, 0)),
            pl.BlockSpec((_NBLK, 8), lambda i: (i, 0)),
            pl.BlockSpec((_NBLK, _H), lambda i: (i, 0)),
            pl.BlockSpec((1, 1, _NBLK), lambda i: (i, 0, 0)),
            wspec((_B, _H)),
            wspec((_H, _H)), wspec((_H, _H)), wspec((1, _H)),
            wspec((_H, _H)), wspec((1, _H)), wspec((1, _H)), wspec((1, _H)),
        ],
        out_specs=[
            pl.BlockSpec((_NBLK, _H), lambda i: (i, 0)),
            pl.BlockSpec((_B, _H), lambda i: (0, 0)),
            pl.BlockSpec((_B, 1), lambda i: (0, 0)),
        ],
        out_shape=[
            jax.ShapeDtypeStruct((_N, _H), _f32),
            jax.ShapeDtypeStruct((_B, _H), _f32),
            jax.ShapeDtypeStruct((_B, 1), _f32),
        ],
    )(p0, p1, c0, c1, uu, batchT, u, *nw)


def _glob_body(ub, gsb, gcb, W1a, W1b, b1, W2, b2, gm, bt, uo):
    mean = gsb[...] / jnp.maximum(gcb[...], 1.0)
    h1 = _gelu(jnp.dot(ub[...], W1a[...], preferred_element_type=_f32)
               + jnp.dot(mean, W1b[...], preferred_element_type=_f32)
               + b1[...])
    uo[...] = _mlp_ln(h1, W2[...], b2[...], gm[...], bt[...])


def _make_glob(u, gs, gc, gw):
    return pl.pallas_call(
        _glob_body,
        out_shape=jax.ShapeDtypeStruct((_B, _H), _f32),
    )(u, gs, gc, *gw)


def kernel(x, edge_index, edge_attr, u, batch,
           e_W1, e_b1, e_W2, e_b2, e_gm, e_bt,
           n1_W1, n1_b1, n1_W2, n1_b2, n1_gm, n1_bt,
           n2_W1, n2_b1, n2_W2, n2_b2, n2_gm, n2_bt,
           g_W1, g_b1, g_W2, g_b2, g_gm, g_bt):
    row = edge_index[0]
    col = edge_index[1]
    pad = _E_PAD - _E
    colg = jnp.concatenate([col, jnp.zeros((pad,), jnp.int32)]).reshape(-1, _CHUNK)
    rows_sc = jnp.concatenate([row, jnp.full((pad,), _N, jnp.int32)]).reshape(-1, _CHUNK)
    batch2d = batch.reshape(_N, 1)
    batchT = batch.reshape(_N // _NBLK, 1, _NBLK)
    r2 = lambda v: v.reshape(1, _H)

    zcnt = jnp.zeros((_N_PAD, 8), _f32)
    onesb = jnp.ones((_CHUNK, 8), _f32)
    a_tab, uu = _make_prep(x, batch2d, u, e_W1[0:_H], e_W1[3 * _H:4 * _H])
    asrc, dstg, c0, c1 = _sc_gather(a_tab, x, rows_sc, colg, zcnt, onesb)

    eye4 = jnp.eye(4, dtype=_f32)
    bd = lambda w: jnp.kron(eye4, w)
    t4 = lambda v: jnp.tile(v.reshape(1, _H), (1, 4))
    Mavg = jnp.kron(eye4, jnp.full((_H, _H), 1.0 / _H, _f32))

    ew = (bd(e_W1[_H:2 * _H]), bd(e_W1[2 * _H:3 * _H]), t4(e_b1),
          bd(e_W2), t4(e_b2), t4(e_gm), t4(e_bt))
    nw = (bd(n1_W1[0:_H]), bd(n1_W1[_H:2 * _H]), t4(n1_b1),
          bd(n1_W2), t4(n1_b2), t4(n1_gm), t4(n1_bt))
    asrc4 = asrc.reshape(_E_PAD // 4, 4 * _H)
    dstg4 = dstg.reshape(_E_PAD // 4, 4 * _H)
    ea4 = edge_attr.reshape(_E // 4, 4 * _H)
    eo4, m4 = _make_edge(asrc4, dstg4, ea4, ew, nw, Mavg)
    edge_out = eo4.reshape(_E, _H)
    m = m4.reshape(_E_PAD, _H)

    zsum = jnp.zeros((_N_PAD, _H), _f32)
    s0, s1 = _sc_scatter(m, rows_sc, zsum)

    n2w = (n2_W1[0:_H], n2_W1[_H:2 * _H], r2(n2_b1),
           n2_W2, r2(n2_b2), r2(n2_gm), r2(n2_bt))
    x_new, gs, gc = _make_node(s0, s1, c0, c1, uu, batchT, u, n2w)

    gw = (g_W1[0:_H], g_W1[_H:2 * _H], r2(g_b1),
          g_W2, r2(g_b2), r2(g_gm), r2(g_bt))
    u_new = _make_glob(u, gs, gc, gw)

    return (x_new, edge_out, u_new)
